# Initial kernel scaffold; baseline (speedup 1.0000x reference)
import jax
import jax.numpy as jnp
from jax.experimental import pallas as pl


def kernel(indices, table):
    raise NotImplementedError("write your pallas kernel here")


# speedup vs baseline: 1.4367x; 1.4367x over previous
import functools

import jax
import jax.numpy as jnp
from jax import lax
from jax.experimental import pallas as pl
from jax.experimental.pallas import tpu as pltpu
from jax.experimental.pallas import tpu_sc as plsc

EMB = 32
CHUNK = 128
NUM_WORKERS = 32


@functools.cache
def _build(n_rows):
    chunks_per_w = n_rows // (CHUNK * NUM_WORKERS)
    mesh = plsc.VectorSubcoreMesh(core_axis_name="c", subcore_axis_name="s")

    @functools.partial(
        pl.kernel,
        mesh=mesh,
        out_type=jax.ShapeDtypeStruct((n_rows, EMB), jnp.float32),
        scratch_types=[
            pltpu.VMEM((chunks_per_w, CHUNK), jnp.int32),
            pltpu.VMEM((CHUNK, EMB), jnp.float32),
            pltpu.SemaphoreType.DMA,
        ],
        compiler_params=pltpu.CompilerParams(use_tc_tiling_on_sc=False),
    )
    def gather_kernel(idx_hbm, table_hbm, out_hbm, idx_v, rows_v, sem):
        wid = lax.axis_index("s") * 2 + lax.axis_index("c")
        base_chunk = wid * chunks_per_w
        pltpu.sync_copy(idx_hbm.at[pl.ds(base_chunk, chunks_per_w)], idx_v)

        def body(g, carry):
            pltpu.async_copy(table_hbm.at[idx_v.at[g]], rows_v, sem).wait()
            pltpu.sync_copy(rows_v, out_hbm.at[pl.ds((base_chunk + g) * CHUNK, CHUNK)])
            return carry

        lax.fori_loop(0, chunks_per_w, body, 0)

    return gather_kernel


def kernel(indices, table):
    n_rows = indices.size
    idx2d = indices.reshape(n_rows // CHUNK, CHUNK).astype(jnp.int32)
    out = _build(n_rows)(idx2d, table)
    return out.reshape(indices.shape + (table.shape[1],))


# speedup vs baseline: 1.5746x; 1.0960x over previous
import functools

import jax
import jax.numpy as jnp
from jax import lax
from jax.experimental import pallas as pl
from jax.experimental.pallas import tpu as pltpu
from jax.experimental.pallas import tpu_sc as plsc

EMB = 32
CHUNK = 128
GROUP = 8
NUM_WORKERS = 32


@functools.cache
def _build(n_rows):
    chunks_per_w = n_rows // (CHUNK * NUM_WORKERS)
    assert chunks_per_w % GROUP == 0
    groups_per_w = chunks_per_w // GROUP
    mesh = plsc.VectorSubcoreMesh(core_axis_name="c", subcore_axis_name="s")

    @functools.partial(
        pl.kernel,
        mesh=mesh,
        out_type=jax.ShapeDtypeStruct((n_rows, EMB), jnp.float32),
        scratch_types=[
            pltpu.VMEM((chunks_per_w, CHUNK), jnp.int32),
            pltpu.VMEM((2, GROUP * CHUNK, EMB), jnp.float32),
            pltpu.SemaphoreType.DMA((2,)),
            pltpu.SemaphoreType.DMA((2,)),
        ],
        compiler_params=pltpu.CompilerParams(use_tc_tiling_on_sc=False),
    )
    def gather_kernel(idx_hbm, table_hbm, out_hbm, idx_v, rows_v, gsem, osem):
        wid = lax.axis_index("s") * 2 + lax.axis_index("c")
        base_chunk = wid * chunks_per_w
        pltpu.sync_copy(idx_hbm.at[pl.ds(base_chunk, chunks_per_w)], idx_v)

        def fire(t, b):
            for j in range(GROUP):
                pltpu.async_copy(
                    table_hbm.at[idx_v.at[t * GROUP + j]],
                    rows_v.at[b, pl.ds(j * CHUNK, CHUNK)],
                    gsem.at[b],
                )

        def drain_gathers(b):
            for j in range(GROUP):
                pltpu.make_async_copy(
                    table_hbm.at[idx_v.at[j]],
                    rows_v.at[b, pl.ds(j * CHUNK, CHUNK)],
                    gsem.at[b],
                ).wait()

        def store_out(t, b):
            return pltpu.async_copy(
                rows_v.at[b],
                out_hbm.at[pl.ds((base_chunk + t * GROUP) * CHUNK, GROUP * CHUNK)],
                osem.at[b],
            )

        def drain_store(t, b):
            pltpu.make_async_copy(
                rows_v.at[b],
                out_hbm.at[pl.ds((base_chunk + t * GROUP) * CHUNK, GROUP * CHUNK)],
                osem.at[b],
            ).wait()

        fire(0, 0)

        def body(t, carry):
            b = t % 2
            @pl.when(t + 1 < groups_per_w)
            def _():
                @pl.when(t >= 1)
                def _():
                    drain_store(t - 1, 1 - b)

                fire(t + 1, 1 - b)

            drain_gathers(b)
            store_out(t, b)
            return carry

        lax.fori_loop(0, groups_per_w, body, 0)
        drain_store(groups_per_w - 2, groups_per_w % 2)
        drain_store(groups_per_w - 1, (groups_per_w - 1) % 2)

    return gather_kernel


def kernel(indices, table):
    n_rows = indices.size
    idx2d = indices.reshape(n_rows // CHUNK, CHUNK).astype(jnp.int32)
    out = _build(n_rows)(idx2d, table)
    return out.reshape(indices.shape + (table.shape[1],))
